# Initial kernel scaffold; baseline (speedup 1.0000x reference)
import jax
import jax.numpy as jnp
from jax.experimental import pallas as pl


def kernel(edge_index, h, r, norm, embed, W0, loop_w0, bias0, W1, loop_w1, bias1):
    raise NotImplementedError("write your pallas kernel here")


# speedup vs baseline: 11.2014x; 11.2014x over previous
import functools

import jax
import jax.numpy as jnp
from jax import lax
from jax.experimental import pallas as pl
from jax.experimental.pallas import tpu as pltpu
from jax.experimental.pallas import tpu_sc as plsc

N = 10000
E = 160000
D = 256
NB = 2
B = D // NB
R = 16
RP1 = R + 1

NSC = 2
NT = 16
EC = 128
NCH = 80
ET = EC * NCH
E_PAD = ET * NT
EROWS = E_PAD // EC
N_PAD = 10240
NPT = N_PAD // NT

TN = 1000
NI = N // TN


def _expand_body(x_ref, wd_ref, y_ref):
    rel = pl.program_id(1)
    y_ref[0] = jnp.dot(x_ref[...], wd_ref[rel],
                       preferred_element_type=jnp.float32)


def _tc_expand(x, wd):
    return pl.pallas_call(
        _expand_body,
        grid=(NI, RP1),
        in_specs=[
            pl.BlockSpec((TN, D), lambda i, rl: (i, 0)),
            pl.BlockSpec((RP1, D, D), lambda i, rl: (0, 0, 0)),
        ],
        out_specs=pl.BlockSpec((1, TN, D), lambda i, rl: (rl, i, 0)),
        out_shape=jax.ShapeDtypeStruct((RP1, N, D), jnp.float32),
    )(x, wd)


def _sc_agg_body(yflat, meta, metaf, out,
                 agg_sh, ring, ringf, rb0, rb1,
                 g0, g1, m0, m1, m2, m3):
    c = lax.axis_index("c")
    s = lax.axis_index("s")
    lo = s * NPT
    j0 = s * NCH

    def zrow(i, carry):
        zv = jnp.zeros((16,), jnp.float32)
        for k in range(B // 16):
            rb0[i, pl.ds(k * 16, 16)] = zv
        return carry
    lax.fori_loop(0, EC, zrow, 0)
    for q in range(NPT // EC):
        pltpu.sync_copy(rb0, agg_sh.at[pl.ds(lo + q * EC, EC)])
    plsc.subcore_barrier()

    gbufs = (rb0, rb1)
    gsems = (g0, g1)
    msems = (m0, m1, m2, m3)

    def fire_meta(j, slot):
        pltpu.make_async_copy(meta.at[c, j0 + j], ring.at[slot],
                              msems[slot]).start()
        pltpu.make_async_copy(metaf.at[j0 + j], ringf.at[slot],
                              msems[slot]).start()

    def wait_meta(slot):
        pltpu.make_async_copy(meta.at[c, j0], ring.at[slot],
                              msems[slot]).wait()
        pltpu.make_async_copy(metaf.at[j0], ringf.at[slot],
                              msems[slot]).wait()

    def fire_gather(slot, b):
        pltpu.make_async_copy(yflat.at[ring.at[slot, 0]], gbufs[b],
                              gsems[b]).start()

    def wait_gather(b):
        pltpu.make_async_copy(yflat.at[ring.at[0, 0]], gbufs[b],
                              gsems[b]).wait()

    for slot in range(4):
        fire_meta(slot, slot)
    for b in range(2):
        wait_meta(b)
        fire_gather(b, b)

    def quad(t, carry):
        for b4 in range(4):
            j = t * 4 + b4
            b = b4 % 2
            rb = gbufs[b]
            wait_gather(b)

            def grp(gg, rcarry):
                nvv = ringf[b4, pl.ds(gg * 16, 16)]
                for g16 in range(16):
                    bc = lax.gather(
                        nvv, jnp.full((16, 1), g16, jnp.int32),
                        lax.GatherDimensionNumbers(
                            offset_dims=(), collapsed_slice_dims=(0,),
                            start_index_map=(0,)),
                        (1,), mode=lax.GatherScatterMode.PROMISE_IN_BOUNDS)
                    row = gg * 16 + g16
                    for k in range(B // 16):
                        rb[row, pl.ds(k * 16, 16)] = (
                            rb[row, pl.ds(k * 16, 16)] * bc)
                return rcarry
            lax.fori_loop(0, EC // 16, grp, 0)

            pltpu.sync_copy(rb, agg_sh.at[ring.at[b4, 1]], add=True)

            @pl.when(j + 4 < NCH)
            def _():
                fire_meta(j + 4, b4)

            @pl.when(j + 2 < NCH)
            def _():
                wait_meta((b4 + 2) % 4)
                fire_gather((b4 + 2) % 4, b)
        return carry
    lax.fori_loop(0, NCH // 4, quad, 0)

    plsc.subcore_barrier()
    for q in range(NPT // EC):
        pltpu.sync_copy(agg_sh.at[pl.ds(lo + q * EC, EC)],
                        out.at[c, pl.ds(lo + q * EC, EC)])


def _sc_agg(yflat, meta, metaf):
    mesh = plsc.VectorSubcoreMesh(core_axis_name="c", subcore_axis_name="s")
    f = pl.kernel(
        _sc_agg_body,
        out_type=jax.ShapeDtypeStruct((NB, N_PAD, B), jnp.float32),
        mesh=mesh,
        scratch_types=[
            pltpu.VMEM_SHARED((N_PAD, B), jnp.float32),
            pltpu.VMEM((4, 2, EC), jnp.int32),
            pltpu.VMEM((4, EC), jnp.float32),
            pltpu.VMEM((EC, B), jnp.float32),
            pltpu.VMEM((EC, B), jnp.float32),
            pltpu.SemaphoreType.DMA,
            pltpu.SemaphoreType.DMA,
            pltpu.SemaphoreType.DMA,
            pltpu.SemaphoreType.DMA,
            pltpu.SemaphoreType.DMA,
            pltpu.SemaphoreType.DMA,
        ],
    )
    return f(yflat, meta, metaf)


def _combine_body(y_ref, a0_ref, a1_ref, b_ref, o_ref, *, act):
    acc = (y_ref[0]
           + jnp.concatenate([a0_ref[0], a1_ref[0]], axis=-1)
           + b_ref[...])
    o_ref[...] = jnp.maximum(acc, 0.0) if act else acc


def _tc_combine(y, agg, bias2, act):
    return pl.pallas_call(
        functools.partial(_combine_body, act=act),
        grid=(NI,),
        in_specs=[
            pl.BlockSpec((1, TN, D), lambda i: (R, i, 0)),
            pl.BlockSpec((1, TN, B), lambda i: (0, i, 0)),
            pl.BlockSpec((1, TN, B), lambda i: (1, i, 0)),
            pl.BlockSpec((1, D), lambda i: (0, 0)),
        ],
        out_specs=pl.BlockSpec((TN, D), lambda i: (i, 0)),
        out_shape=jax.ShapeDtypeStruct((N, D), jnp.float32),
    )(y, agg, agg, bias2)


def _layer(x, meta, metaf, W, loop_w, bias, act):
    wd = jnp.zeros((RP1, D, D), jnp.float32)
    wd = (wd.at[:R, :B, :B].set(W[:, 0])
            .at[:R, B:, B:].set(W[:, 1])
            .at[R].set(loop_w))
    y = _tc_expand(x, wd)
    agg = _sc_agg(y.reshape(RP1 * N * NB, B), meta, metaf)
    return _tc_combine(y, agg, bias.reshape(1, D), act)


def kernel(edge_index, h, r, norm, embed,
           W0, loop_w0, bias0, W1, loop_w1, bias1):
    src = edge_index[0].astype(jnp.int32)
    dst = edge_index[1].astype(jnp.int32)
    x = jnp.take(embed, h, axis=0)

    base = (r.astype(jnp.int32) * N + src) * NB
    pad = E_PAD - E
    dstp = jnp.pad(dst, (0, pad))
    meta = jnp.stack([
        jnp.stack([jnp.pad(base, (0, pad)), dstp]),
        jnp.stack([jnp.pad(base + 1, (0, pad)), dstp]),
    ])
    meta = meta.reshape(NB, 2, EROWS, EC).transpose(0, 2, 1, 3)
    metaf = jnp.pad(norm[:, 0], (0, pad)).reshape(EROWS, EC)

    x = _layer(x, meta, metaf, W0, loop_w0, bias0, True)
    x = _layer(x, meta, metaf, W1, loop_w1, bias1, False)
    return x


# speedup vs baseline: 15.1060x; 1.3486x over previous
import functools

import jax
import jax.numpy as jnp
from jax import lax
from jax.experimental import pallas as pl
from jax.experimental.pallas import tpu as pltpu
from jax.experimental.pallas import tpu_sc as plsc

N = 10000
E = 160000
D = 256
NB = 2
B = D // NB
R = 16
RP1 = R + 1

NSC = 2
NT = 16
EC = 112
NCH = 90
ET = EC * NCH
E_PAD = ET * NT
EROWS = E_PAD // EC
N_PAD = 10112
NPT = N_PAD // NT
NTAIL = NPT - (NPT // EC) * EC

TN = 1000
NI = N // TN


def _expand_body(x_ref, wd_ref, y_ref):
    rel = pl.program_id(1)
    y_ref[0] = jnp.dot(x_ref[...], wd_ref[rel],
                       preferred_element_type=jnp.float32)


def _tc_expand(x, wd):
    return pl.pallas_call(
        _expand_body,
        grid=(NI, RP1),
        in_specs=[
            pl.BlockSpec((TN, D), lambda i, rl: (i, 0)),
            pl.BlockSpec((RP1, D, D), lambda i, rl: (0, 0, 0)),
        ],
        out_specs=pl.BlockSpec((1, TN, D), lambda i, rl: (rl, i, 0)),
        out_shape=jax.ShapeDtypeStruct((RP1, N, D), jnp.float32),
    )(x, wd)


def _sc_agg_body(yflat, meta, metaf, out,
                 agg_sh, ring, ringf, dstb, rb0, rb1, rb2,
                 g0, g1, g2, s0, s1, s2, m0, m1, m2):
    c = lax.axis_index("c")
    s = lax.axis_index("s")
    lo = s * NPT
    j0 = s * NCH

    def zrow(i, carry):
        zv = jnp.zeros((16,), jnp.float32)
        for k in range(B // 16):
            rb0[i, pl.ds(k * 16, 16)] = zv
        return carry
    lax.fori_loop(0, EC, zrow, 0)
    for q in range(NPT // EC):
        pltpu.sync_copy(rb0, agg_sh.at[pl.ds(lo + q * EC, EC)])
    pltpu.sync_copy(rb0.at[pl.ds(0, NTAIL)],
                    agg_sh.at[pl.ds(lo + (NPT // EC) * EC, NTAIL)])
    plsc.subcore_barrier()

    gbufs = (rb0, rb1, rb2)
    gsems = (g0, g1, g2)
    ssems = (s0, s1, s2)
    msems = (m0, m1, m2)

    def fire_meta(j, slot):
        pltpu.make_async_copy(meta.at[c, j0 + j], ring.at[slot],
                              msems[slot]).start()
        pltpu.make_async_copy(metaf.at[j0 + j], ringf.at[slot],
                              msems[slot]).start()

    def wait_meta(slot):
        pltpu.make_async_copy(meta.at[c, j0], ring.at[slot],
                              msems[slot]).wait()
        pltpu.make_async_copy(metaf.at[j0], ringf.at[slot],
                              msems[slot]).wait()

    def fire_gather(b):
        pltpu.make_async_copy(yflat.at[ring.at[b, 0]], gbufs[b],
                              gsems[b]).start()

    def wait_gather(b):
        pltpu.make_async_copy(yflat.at[ring.at[0, 0]], gbufs[b],
                              gsems[b]).wait()

    def wait_scatter(b):
        pltpu.make_async_copy(gbufs[b], agg_sh.at[dstb.at[b]],
                              ssems[b]).wait()

    for slot in range(3):
        fire_meta(slot, slot)
    for b in range(2):
        wait_meta(b)
        fire_gather(b)

    def trip(t, carry):
        for b in range(3):
            j = t * 3 + b
            rb = gbufs[b]
            wait_gather(b)

            def grp(gg, rcarry):
                nvv = ringf[b, pl.ds(gg * 16, 16)]
                for g16 in range(16):
                    bc = lax.gather(
                        nvv, jnp.full((16, 1), g16, jnp.int32),
                        lax.GatherDimensionNumbers(
                            offset_dims=(), collapsed_slice_dims=(0,),
                            start_index_map=(0,)),
                        (1,), mode=lax.GatherScatterMode.PROMISE_IN_BOUNDS)
                    row = gg * 16 + g16
                    for k in range(B // 16):
                        rb[row, pl.ds(k * 16, 16)] = (
                            rb[row, pl.ds(k * 16, 16)] * bc)
                return rcarry
            lax.fori_loop(0, EC // 16, grp, 0)

            for k in range(EC // 16):
                dstb[b, pl.ds(k * 16, 16)] = ring[b, 1, pl.ds(k * 16, 16)]
            pltpu.async_copy(rb, agg_sh.at[dstb.at[b]], ssems[b], add=True)

            @pl.when(j + 3 < NCH)
            def _():
                fire_meta(j + 3, b)

            @pl.when(j >= 1)
            def _():
                wait_scatter((b + 2) % 3)

            @pl.when(j + 2 < NCH)
            def _():
                wait_meta((b + 2) % 3)
                fire_gather((b + 2) % 3)
        return carry
    lax.fori_loop(0, NCH // 3, trip, 0)
    wait_scatter((NCH - 1) % 3)

    plsc.subcore_barrier()
    for q in range(NPT // EC):
        pltpu.sync_copy(agg_sh.at[pl.ds(lo + q * EC, EC)],
                        out.at[c, pl.ds(lo + q * EC, EC)])
    pltpu.sync_copy(agg_sh.at[pl.ds(lo + (NPT // EC) * EC, NTAIL)],
                    out.at[c, pl.ds(lo + (NPT // EC) * EC, NTAIL)])


def _sc_agg(yflat, meta, metaf):
    mesh = plsc.VectorSubcoreMesh(core_axis_name="c", subcore_axis_name="s")
    f = pl.kernel(
        _sc_agg_body,
        out_type=jax.ShapeDtypeStruct((NB, N_PAD, B), jnp.float32),
        mesh=mesh,
        scratch_types=[
            pltpu.VMEM_SHARED((N_PAD, B), jnp.float32),
            pltpu.VMEM((3, 2, EC), jnp.int32),
            pltpu.VMEM((3, EC), jnp.float32),
            pltpu.VMEM((3, EC), jnp.int32),
            pltpu.VMEM((EC, B), jnp.float32),
            pltpu.VMEM((EC, B), jnp.float32),
            pltpu.VMEM((EC, B), jnp.float32),
        ] + [pltpu.SemaphoreType.DMA] * 9,
    )
    return f(yflat, meta, metaf)


def _combine_body(y_ref, a0_ref, a1_ref, b_ref, o_ref, *, act):
    acc = (y_ref[0]
           + jnp.concatenate([a0_ref[0], a1_ref[0]], axis=-1)
           + b_ref[...])
    o_ref[...] = jnp.maximum(acc, 0.0) if act else acc


def _tc_combine(y, agg, bias2, act):
    return pl.pallas_call(
        functools.partial(_combine_body, act=act),
        grid=(NI,),
        in_specs=[
            pl.BlockSpec((1, TN, D), lambda i: (R, i, 0)),
            pl.BlockSpec((1, TN, B), lambda i: (0, i, 0)),
            pl.BlockSpec((1, TN, B), lambda i: (1, i, 0)),
            pl.BlockSpec((1, D), lambda i: (0, 0)),
        ],
        out_specs=pl.BlockSpec((TN, D), lambda i: (i, 0)),
        out_shape=jax.ShapeDtypeStruct((N, D), jnp.float32),
    )(y, agg, agg, bias2)


def _layer(x, meta, metaf, W, loop_w, bias, act):
    wd = jnp.zeros((RP1, D, D), jnp.float32)
    wd = (wd.at[:R, :B, :B].set(W[:, 0])
            .at[:R, B:, B:].set(W[:, 1])
            .at[R].set(loop_w))
    y = _tc_expand(x, wd)
    agg = _sc_agg(y.reshape(RP1 * N * NB, B), meta, metaf)
    return _tc_combine(y, agg, bias.reshape(1, D), act)


def kernel(edge_index, h, r, norm, embed,
           W0, loop_w0, bias0, W1, loop_w1, bias1):
    src = edge_index[0].astype(jnp.int32)
    dst = edge_index[1].astype(jnp.int32)
    x = jnp.take(embed, h, axis=0)

    base = (r.astype(jnp.int32) * N + src) * NB
    pad = E_PAD - E
    dstp = jnp.pad(dst, (0, pad))
    meta = jnp.stack([
        jnp.stack([jnp.pad(base, (0, pad)), dstp]),
        jnp.stack([jnp.pad(base + 1, (0, pad)), dstp]),
    ])
    meta = meta.reshape(NB, 2, EROWS, EC).transpose(0, 2, 1, 3)
    metaf = jnp.pad(norm[:, 0], (0, pad)).reshape(EROWS, EC)

    x = _layer(x, meta, metaf, W0, loop_w0, bias0, True)
    x = _layer(x, meta, metaf, W1, loop_w1, bias1, False)
    return x
